# baseline (speedup 1.0000x reference)
import functools

import jax
import jax.numpy as jnp
from jax import lax
from jax.experimental import pallas as pl
from jax.experimental.pallas import tpu as pltpu
from jax.experimental.pallas import tpu_sc as plsc

_NUM_CENTERS = 1000000
_DIM = 64
_BATCH = 16384

_info = plsc.get_sparse_core_info()
_NC, _NS = _info.num_cores, _info.num_subcores
_NW = _NC * _NS
_BPW = _BATCH // _NW
_CHUNK = 128
_NCHUNK = _BPW // _CHUNK

_mesh = plsc.VectorSubcoreMesh(core_axis_name="c", subcore_axis_name="s")


@functools.partial(
    pl.kernel,
    mesh=_mesh,
    out_type=jax.ShapeDtypeStruct((_BATCH, _DIM), jnp.float32),
    scratch_types=[
        pltpu.VMEM((_NCHUNK, _CHUNK), jnp.int32),
        pltpu.VMEM((_BPW, _DIM), jnp.float32),
        pltpu.SemaphoreType.DMA,
    ],
    compiler_params=pltpu.CompilerParams(use_tc_tiling_on_sc=False),
)
def _gather_kernel(idx_hbm, table_hbm, out_hbm, idx_v, rows_v, sem):
    wid = lax.axis_index("s") * _NC + lax.axis_index("c")
    base = wid * _NCHUNK
    pltpu.sync_copy(idx_hbm.at[pl.ds(base, _NCHUNK)], idx_v)
    copies = []
    for j in range(_NCHUNK):
        copies.append(
            pltpu.async_copy(
                table_hbm.at[idx_v.at[j]],
                rows_v.at[pl.ds(j * _CHUNK, _CHUNK)],
                sem,
            )
        )
    for c in copies:
        c.wait()
    pltpu.sync_copy(rows_v, out_hbm.at[pl.ds(wid * _BPW, _BPW)])


def kernel(idx, stored_mean):
    idx2 = idx.reshape(_NW * _NCHUNK, _CHUNK)
    return _gather_kernel(idx2, stored_mean)


# speedup vs baseline: 4.1265x; 4.1265x over previous
import functools

import jax
import jax.numpy as jnp
from jax import lax
from jax.experimental import pallas as pl
from jax.experimental.pallas import tpu as pltpu
from jax.experimental.pallas import tpu_sc as plsc

_NUM_CENTERS = 1000000
_DIM = 64
_BATCH = 16384
_LANES = 128
_LAST_RC = (_NUM_CENTERS - 1) // _LANES

_info = plsc.get_sparse_core_info()
_NC, _NS = _info.num_cores, _info.num_subcores
_NW = _NC * _NS
_RPW = _BATCH // _NW
_NBUF = 4
_BLK = 64

_mesh = plsc.VectorSubcoreMesh(core_axis_name="c", subcore_axis_name="s")


def _worker_id():
    return lax.axis_index("s") * _NC + lax.axis_index("c")


def _iota16():
    return lax.iota(jnp.int32, 16)


def _sload(ref, i):
    g = pl.multiple_of((i // 16) * 16, 8)
    grp = ref[pl.ds(g, 16)]
    return jnp.sum(jnp.where(_iota16() == i - g, grp, 0))


def _sstore(ref, i, val):
    plsc.store_scatter(
        ref,
        [jnp.full((16,), i, jnp.int32)],
        jnp.full((16,), val, jnp.int32),
        mask=_iota16() == 0,
    )


@functools.partial(
    pl.kernel,
    mesh=_mesh,
    out_type=jax.ShapeDtypeStruct((_BATCH, _LANES), jnp.float32),
    scratch_types=[
        pltpu.VMEM((_RPW,), jnp.int32),
        pltpu.VMEM((_RPW + 16,), jnp.int32),
        pltpu.VMEM((_NBUF, _DIM, _LANES), jnp.float32),
        pltpu.VMEM((2, _BLK, _LANES), jnp.float32),
        pltpu.SemaphoreType.DMA((_NBUF,)),
        pltpu.SemaphoreType.DMA,
    ],
    compiler_params=pltpu.CompilerParams(
        use_tc_tiling_on_sc=True, needs_layout_passes=False),
)
def _stage1(sorted_hbm, tableT_hbm, tail_hbm, inter_hbm,
            s_v, runs_v, slab_v, rbuf, gsem, osem):
    w = _worker_id()
    base = w * _RPW
    iota16 = _iota16()
    pltpu.sync_copy(sorted_hbm.at[pl.ds(base, _RPW)], s_v)

    def scan_body(g, carry):
        off, prev = carry
        vrc = s_v[pl.ds(g * 16, 16)] >> 7
        sh = jnp.take(vrc, jnp.maximum(iota16 - 1, 0))
        pr = jnp.where(iota16 == 0, prev, sh)
        is_new = vrc != pr
        plsc.store_compressed(
            runs_v.at[pl.ds(off, 16)], iota16 + 16 * g, mask=is_new)
        cnt = jnp.max(plsc.all_reduce_population_count(is_new))
        last = jnp.take(vrc, jnp.full((16,), 15, jnp.int32))
        return off + cnt, last

    nrun, _ = lax.fori_loop(
        0, _RPW // 16, scan_body,
        (0, jnp.full((16,), -1, jnp.int32)))
    _sstore(runs_v, nrun, _RPW)

    def fetch(r, buf):
        rc = _sload(s_v, _sload(runs_v, r)) >> 7

        @pl.when(rc < _LAST_RC)
        def _():
            start = pl.multiple_of(rc * _LANES, _LANES)
            pltpu.async_copy(
                tableT_hbm.at[:, pl.ds(start, _LANES)],
                slab_v.at[buf], gsem.at[buf])

        @pl.when(rc >= _LAST_RC)
        def _():
            pltpu.async_copy(tail_hbm, slab_v.at[buf], gsem.at[buf])

    for k in range(_NBUF - 1):
        @pl.when(k < nrun)
        def _(k=k):
            fetch(k, k)

    def run_body(r, _):
        buf = lax.rem(r, _NBUF)
        pltpu.make_async_copy(
            tableT_hbm.at[:, pl.ds(0, _LANES)], slab_v.at[buf], gsem.at[buf]
        ).wait()
        r2 = r + _NBUF - 1

        @pl.when(r2 < nrun)
        def _():
            fetch(r2, lax.rem(r2, _NBUF))

        def row_body(q, _):
            par = lax.rem(q // _BLK, 2)
            qm = lax.rem(q, _BLK)

            @pl.when(jnp.logical_and(qm == 0, q >= 2 * _BLK))
            def _():
                pltpu.make_async_copy(
                    rbuf.at[0], inter_hbm.at[pl.ds(0, _BLK), :], osem
                ).wait()

            g = pl.multiple_of((q // 16) * 16, 8)
            grp_rl = s_v[pl.ds(g, 16)] & (_LANES - 1)
            rlv = jnp.take(grp_rl, jnp.full((16,), q - g, jnp.int32))
            for k in range(_DIM // 16):
                vals = plsc.load_gather(
                    slab_v.at[buf], [iota16 + (16 * k), rlv])
                rbuf[par, qm, pl.ds(16 * k, 16)] = vals

            @pl.when(qm == _BLK - 1)
            def _():
                b = q // _BLK
                pltpu.async_copy(
                    rbuf.at[par],
                    inter_hbm.at[pl.ds(base + b * _BLK, _BLK), :],
                    osem)

            return 0

        lax.fori_loop(_sload(runs_v, r), _sload(runs_v, r + 1), row_body, 0)
        return 0

    lax.fori_loop(0, nrun, run_body, 0)
    for _ in range(2):
        pltpu.make_async_copy(
            rbuf.at[0], inter_hbm.at[pl.ds(0, _BLK), :], osem
        ).wait()


@functools.partial(
    pl.kernel,
    mesh=_mesh,
    out_type=jax.ShapeDtypeStruct((_BATCH, _LANES), jnp.float32),
    scratch_types=[
        pltpu.VMEM((_LANES,), jnp.int32),
        pltpu.VMEM((2, _LANES, _LANES), jnp.float32),
        pltpu.SemaphoreType.DMA((2,)),
        pltpu.SemaphoreType.DMA,
    ],
    compiler_params=pltpu.CompilerParams(
        use_tc_tiling_on_sc=True, needs_layout_passes=False),
)
def _stage2(inv_hbm, inter_hbm, out_hbm, idx_v, gbuf, gsem, osem):
    w = _worker_id()
    base = w * _RPW
    nchunk = _RPW // _LANES
    for t in range(nchunk):
        p = t % 2
        if t >= 2:
            pltpu.make_async_copy(
                gbuf.at[p], out_hbm.at[pl.ds(0, _LANES), :], osem
            ).wait()
        pltpu.sync_copy(inv_hbm.at[pl.ds(base + t * _LANES, _LANES)], idx_v)
        pltpu.async_copy(inter_hbm.at[idx_v], gbuf.at[p], gsem.at[p]).wait()
        pltpu.async_copy(
            gbuf.at[p],
            out_hbm.at[pl.ds(base + t * _LANES, _LANES), :],
            osem)
    for _ in range(2):
        pltpu.make_async_copy(
            gbuf.at[0], out_hbm.at[pl.ds(0, _LANES), :], osem
        ).wait()


def kernel(idx, stored_mean):
    ord_ = jnp.argsort(idx).astype(jnp.int32)
    sorted_idx = jnp.take(idx, ord_, axis=0)
    inv = jnp.zeros((_BATCH,), jnp.int32).at[ord_].set(
        jnp.arange(_BATCH, dtype=jnp.int32))
    tableT = stored_mean.T
    tail = jnp.pad(
        tableT[:, _LAST_RC * _LANES:],
        ((0, 0), (0, _LAST_RC * _LANES + _LANES - _NUM_CENTERS)))
    inter = _stage1(sorted_idx, tableT, tail)
    out128 = _stage2(inv, inter)
    return out128[:, :_DIM]


# speedup vs baseline: 5.4028x; 1.3093x over previous
import functools

import jax
import jax.numpy as jnp
from jax import lax
from jax.experimental import pallas as pl
from jax.experimental.pallas import tpu as pltpu
from jax.experimental.pallas import tpu_sc as plsc

_NUM_CENTERS = 1000000
_DIM = 64
_BATCH = 16384
_LANES = 128
_LAST_RC = (_NUM_CENTERS - 1) // _LANES

_info = plsc.get_sparse_core_info()
_NC, _NS = _info.num_cores, _info.num_subcores
_NW = _NC * _NS
_RPW = _BATCH // _NW
_NBUF = 6
_BLK = 64
_NBLK = _RPW // _BLK

_mesh = plsc.VectorSubcoreMesh(core_axis_name="c", subcore_axis_name="s")


def _worker_id():
    return lax.axis_index("s") * _NC + lax.axis_index("c")


def _iota16():
    return lax.iota(jnp.int32, 16)


def _sload(ref, i):
    g = pl.multiple_of((i // 16) * 16, 8)
    grp = ref[pl.ds(g, 16)]
    return jnp.sum(jnp.where(_iota16() == i - g, grp, 0))


def _sstore(ref, i, val):
    plsc.store_scatter(
        ref,
        [jnp.full((16,), i, jnp.int32)],
        jnp.full((16,), val, jnp.int32),
        mask=_iota16() == 0,
    )


@functools.partial(
    pl.kernel,
    mesh=_mesh,
    out_type=jax.ShapeDtypeStruct((_BATCH, _LANES), jnp.float32),
    scratch_types=[
        pltpu.VMEM((_RPW,), jnp.int32),
        pltpu.VMEM((_RPW + 16,), jnp.int32),
        pltpu.VMEM((_NBLK, _BLK), jnp.int32),
        pltpu.VMEM((_NBUF, _DIM, _LANES), jnp.float32),
        pltpu.VMEM((2, _BLK, _LANES), jnp.float32),
        pltpu.SemaphoreType.DMA((_NBUF,)),
        pltpu.SemaphoreType.DMA,
    ],
    compiler_params=pltpu.CompilerParams(
        use_tc_tiling_on_sc=True, needs_layout_passes=False),
)
def _gather1(sorted_hbm, ord_hbm, tableT_hbm, tail_hbm, out_hbm,
             s_v, runs_v, ord_v, slab_v, rbuf, gsem, osem):
    w = _worker_id()
    base = w * _RPW
    iota16 = _iota16()
    pltpu.sync_copy(sorted_hbm.at[pl.ds(base, _RPW)], s_v)
    pltpu.sync_copy(ord_hbm.at[pl.ds(w * _NBLK, _NBLK), :], ord_v)

    def scan_body(g, carry):
        off, prev = carry
        vrc = s_v[pl.ds(g * 16, 16)] >> 7
        sh = jnp.take(vrc, jnp.maximum(iota16 - 1, 0))
        pr = jnp.where(iota16 == 0, prev, sh)
        is_new = vrc != pr
        plsc.store_compressed(
            runs_v.at[pl.ds(off, 16)], iota16 + 16 * g, mask=is_new)
        cnt = jnp.max(plsc.all_reduce_population_count(is_new))
        last = jnp.take(vrc, jnp.full((16,), 15, jnp.int32))
        return off + cnt, last

    nrun, _ = lax.fori_loop(
        0, _RPW // 16, scan_body,
        (0, jnp.full((16,), -1, jnp.int32)))
    _sstore(runs_v, nrun, _RPW)

    def fetch(r, buf):
        rc = _sload(s_v, _sload(runs_v, r)) >> 7

        @pl.when(rc < _LAST_RC)
        def _():
            start = pl.multiple_of(rc * _LANES, _LANES)
            pltpu.async_copy(
                tableT_hbm.at[:, pl.ds(start, _LANES)],
                slab_v.at[buf], gsem.at[buf])

        @pl.when(rc >= _LAST_RC)
        def _():
            pltpu.async_copy(tail_hbm, slab_v.at[buf], gsem.at[buf])

    for k in range(_NBUF - 1):
        @pl.when(k < nrun)
        def _(k=k):
            fetch(k, k)

    def run_body(r, _):
        buf = lax.rem(r, _NBUF)
        r2 = r + _NBUF - 1

        @pl.when(r2 < nrun)
        def _():
            fetch(r2, lax.rem(r2, _NBUF))

        pltpu.make_async_copy(
            tableT_hbm.at[:, pl.ds(0, _LANES)], slab_v.at[buf], gsem.at[buf]
        ).wait()

        def row_body(q, _):
            par = lax.rem(q // _BLK, 2)
            qm = lax.rem(q, _BLK)

            @pl.when(jnp.logical_and(qm == 0, q >= 2 * _BLK))
            def _():
                pltpu.make_async_copy(
                    rbuf.at[0], out_hbm.at[pl.ds(0, _BLK), :], osem
                ).wait()

            g = pl.multiple_of((q // 16) * 16, 8)
            grp_rl = s_v[pl.ds(g, 16)] & (_LANES - 1)
            rlv = jnp.take(grp_rl, jnp.full((16,), q - g, jnp.int32))
            for k in range(_DIM // 16):
                vals = plsc.load_gather(
                    slab_v.at[buf], [iota16 + (16 * k), rlv])
                rbuf[par, qm, pl.ds(16 * k, 16)] = vals

            @pl.when(qm == _BLK - 1)
            def _():
                pltpu.async_copy(
                    rbuf.at[par],
                    out_hbm.at[ord_v.at[q // _BLK]],
                    osem)

            return 0

        lax.fori_loop(_sload(runs_v, r), _sload(runs_v, r + 1), row_body, 0)
        return 0

    lax.fori_loop(0, nrun, run_body, 0)
    for _ in range(2):
        pltpu.make_async_copy(
            rbuf.at[0], out_hbm.at[pl.ds(0, _BLK), :], osem
        ).wait()


def kernel(idx, stored_mean):
    sorted_idx, ord_ = lax.sort(
        (idx, jnp.arange(_BATCH, dtype=jnp.int32)), num_keys=1)
    ord2 = ord_.reshape(_NW * _NBLK, _BLK)
    tableT = stored_mean.T
    tail = jnp.pad(
        tableT[:, _LAST_RC * _LANES:],
        ((0, 0), (0, _LAST_RC * _LANES + _LANES - _NUM_CENTERS)))
    out128 = _gather1(sorted_idx, ord2, tableT, tail)
    return out128[:, :_DIM]


# speedup vs baseline: 5.4257x; 1.0042x over previous
import functools

import jax
import jax.numpy as jnp
from jax import lax
from jax.experimental import pallas as pl
from jax.experimental.pallas import tpu as pltpu
from jax.experimental.pallas import tpu_sc as plsc

_NUM_CENTERS = 1000000
_DIM = 64
_BATCH = 16384
_LANES = 128
_LAST_RC = (_NUM_CENTERS - 1) // _LANES

_info = plsc.get_sparse_core_info()
_NC, _NS = _info.num_cores, _info.num_subcores
_NW = _NC * _NS
_RPW = _BATCH // _NW
_NBUF = 8
_BLK = 64
_NBLK = _RPW // _BLK

_mesh = plsc.VectorSubcoreMesh(core_axis_name="c", subcore_axis_name="s")


def _worker_id():
    return lax.axis_index("s") * _NC + lax.axis_index("c")


def _iota16():
    return lax.iota(jnp.int32, 16)


def _sload(ref, i):
    g = pl.multiple_of((i // 16) * 16, 8)
    grp = ref[pl.ds(g, 16)]
    return jnp.sum(jnp.where(_iota16() == i - g, grp, 0))


def _sstore(ref, i, val):
    plsc.store_scatter(
        ref,
        [jnp.full((16,), i, jnp.int32)],
        jnp.full((16,), val, jnp.int32),
        mask=_iota16() == 0,
    )


@functools.partial(
    pl.kernel,
    mesh=_mesh,
    out_type=jax.ShapeDtypeStruct((_BATCH, _LANES), jnp.float32),
    scratch_types=[
        pltpu.VMEM((_RPW,), jnp.int32),
        pltpu.VMEM((_RPW + 16,), jnp.int32),
        pltpu.VMEM((_NBLK, _BLK), jnp.int32),
        pltpu.VMEM((_NBUF, _DIM, _LANES), jnp.float32),
        pltpu.VMEM((2, _BLK, _LANES), jnp.float32),
        pltpu.SemaphoreType.DMA((_NBUF,)),
        pltpu.SemaphoreType.DMA,
    ],
    compiler_params=pltpu.CompilerParams(
        use_tc_tiling_on_sc=True, needs_layout_passes=False),
)
def _gather1(sorted_hbm, ord_hbm, tableT_hbm, tail_hbm, out_hbm,
             s_v, runs_v, ord_v, slab_v, rbuf, gsem, osem):
    w = _worker_id()
    base = w * _RPW
    iota16 = _iota16()
    pltpu.sync_copy(sorted_hbm.at[pl.ds(base, _RPW)], s_v)
    pltpu.sync_copy(ord_hbm.at[pl.ds(w * _NBLK, _NBLK), :], ord_v)

    def scan_body(g, carry):
        off, prev = carry
        vrc = s_v[pl.ds(g * 16, 16)] >> 7
        sh = jnp.take(vrc, jnp.maximum(iota16 - 1, 0))
        pr = jnp.where(iota16 == 0, prev, sh)
        is_new = vrc != pr
        plsc.store_compressed(
            runs_v.at[pl.ds(off, 16)], iota16 + 16 * g, mask=is_new)
        cnt = jnp.max(plsc.all_reduce_population_count(is_new))
        last = jnp.take(vrc, jnp.full((16,), 15, jnp.int32))
        return off + cnt, last

    nrun, _ = lax.fori_loop(
        0, _RPW // 16, scan_body,
        (0, jnp.full((16,), -1, jnp.int32)))
    _sstore(runs_v, nrun, _RPW)

    def fetch(r, buf):
        rc = _sload(s_v, _sload(runs_v, r)) >> 7

        @pl.when(rc < _LAST_RC)
        def _():
            start = pl.multiple_of(rc * _LANES, _LANES)
            pltpu.async_copy(
                tableT_hbm.at[:, pl.ds(start, _LANES)],
                slab_v.at[buf], gsem.at[buf])

        @pl.when(rc >= _LAST_RC)
        def _():
            pltpu.async_copy(tail_hbm, slab_v.at[buf], gsem.at[buf])

    for k in range(_NBUF - 1):
        @pl.when(k < nrun)
        def _(k=k):
            fetch(k, k)

    def run_body(r, _):
        buf = lax.rem(r, _NBUF)
        r2 = r + _NBUF - 1

        @pl.when(r2 < nrun)
        def _():
            fetch(r2, lax.rem(r2, _NBUF))

        pltpu.make_async_copy(
            tableT_hbm.at[:, pl.ds(0, _LANES)], slab_v.at[buf], gsem.at[buf]
        ).wait()

        def row_body(q, _):
            par = lax.rem(q // _BLK, 2)
            qm = lax.rem(q, _BLK)

            @pl.when(jnp.logical_and(qm == 0, q >= 2 * _BLK))
            def _():
                pltpu.make_async_copy(
                    rbuf.at[0], out_hbm.at[pl.ds(0, _BLK), :], osem
                ).wait()

            g = pl.multiple_of((q // 16) * 16, 8)
            grp_rl = s_v[pl.ds(g, 16)] & (_LANES - 1)
            rlv = jnp.take(grp_rl, jnp.full((16,), q - g, jnp.int32))
            for k in range(_DIM // 16):
                vals = plsc.load_gather(
                    slab_v.at[buf], [iota16 + (16 * k), rlv])
                rbuf[par, qm, pl.ds(16 * k, 16)] = vals

            @pl.when(qm == _BLK - 1)
            def _():
                pltpu.async_copy(
                    rbuf.at[par],
                    out_hbm.at[ord_v.at[q // _BLK]],
                    osem)

            return 0

        lax.fori_loop(_sload(runs_v, r), _sload(runs_v, r + 1), row_body, 0)
        return 0

    lax.fori_loop(0, nrun, run_body, 0)
    for _ in range(2):
        pltpu.make_async_copy(
            rbuf.at[0], out_hbm.at[pl.ds(0, _BLK), :], osem
        ).wait()


def kernel(idx, stored_mean):
    sorted_idx, ord_ = lax.sort(
        (idx, jnp.arange(_BATCH, dtype=jnp.int32)), num_keys=1)
    ord2 = ord_.reshape(_NW * _NBLK, _BLK)
    tableT = stored_mean.T
    tail = jnp.pad(
        tableT[:, _LAST_RC * _LANES:],
        ((0, 0), (0, _LAST_RC * _LANES + _LANES - _NUM_CENTERS)))
    out128 = _gather1(sorted_idx, ord2, tableT, tail)
    return out128[:, :_DIM]


# speedup vs baseline: 5.4420x; 1.0030x over previous
import functools

import jax
import jax.numpy as jnp
from jax import lax
from jax.experimental import pallas as pl
from jax.experimental.pallas import tpu as pltpu
from jax.experimental.pallas import tpu_sc as plsc

_NUM_CENTERS = 1000000
_DIM = 64
_BATCH = 16384
_LANES = 128
_LAST_RC = (_NUM_CENTERS - 1) // _LANES

_info = plsc.get_sparse_core_info()
_NC, _NS = _info.num_cores, _info.num_subcores
_NW = _NC * _NS
_RPW = _BATCH // _NW
_NBUF = 8
_BLK = 64
_NBLK = _RPW // _BLK

_mesh = plsc.VectorSubcoreMesh(core_axis_name="c", subcore_axis_name="s")


def _worker_id():
    return lax.axis_index("s") * _NC + lax.axis_index("c")


def _iota16():
    return lax.iota(jnp.int32, 16)


def _sload(ref, i):
    g = pl.multiple_of((i // 16) * 16, 8)
    grp = ref[pl.ds(g, 16)]
    return jnp.sum(jnp.where(_iota16() == i - g, grp, 0))


def _sstore(ref, i, val):
    plsc.store_scatter(
        ref,
        [jnp.full((16,), i, jnp.int32)],
        jnp.full((16,), val, jnp.int32),
        mask=_iota16() == 0,
    )


@functools.partial(
    pl.kernel,
    mesh=_mesh,
    out_type=jax.ShapeDtypeStruct((_BATCH, _LANES), jnp.float32),
    scratch_types=[
        pltpu.VMEM((_RPW,), jnp.int32),
        pltpu.VMEM((_RPW + 16,), jnp.int32),
        pltpu.VMEM((_NBLK, _BLK), jnp.int32),
        pltpu.VMEM((_NBUF, _DIM, _LANES), jnp.float32),
        pltpu.VMEM((4, _BLK, _LANES), jnp.float32),
        pltpu.SemaphoreType.DMA((_NBUF,)),
        pltpu.SemaphoreType.DMA,
    ],
    compiler_params=pltpu.CompilerParams(
        use_tc_tiling_on_sc=True, needs_layout_passes=False),
)
def _gather1(sorted_hbm, ord_hbm, tableT_hbm, tail_hbm, out_hbm,
             s_v, runs_v, ord_v, slab_v, rbuf, gsem, osem):
    w = _worker_id()
    base = w * _RPW
    iota16 = _iota16()
    pltpu.sync_copy(sorted_hbm.at[pl.ds(base, _RPW)], s_v)
    pltpu.sync_copy(ord_hbm.at[pl.ds(w * _NBLK, _NBLK), :], ord_v)

    def scan_body(g, carry):
        off, prev = carry
        vrc = s_v[pl.ds(g * 16, 16)] >> 7
        sh = jnp.take(vrc, jnp.maximum(iota16 - 1, 0))
        pr = jnp.where(iota16 == 0, prev, sh)
        is_new = vrc != pr
        plsc.store_compressed(
            runs_v.at[pl.ds(off, 16)], iota16 + 16 * g, mask=is_new)
        cnt = jnp.max(plsc.all_reduce_population_count(is_new))
        last = jnp.take(vrc, jnp.full((16,), 15, jnp.int32))
        return off + cnt, last

    nrun, _ = lax.fori_loop(
        0, _RPW // 16, scan_body,
        (0, jnp.full((16,), -1, jnp.int32)))
    _sstore(runs_v, nrun, _RPW)

    def fetch(r, buf):
        rc = _sload(s_v, _sload(runs_v, r)) >> 7

        @pl.when(rc < _LAST_RC)
        def _():
            start = pl.multiple_of(rc * _LANES, _LANES)
            pltpu.async_copy(
                tableT_hbm.at[:, pl.ds(start, _LANES)],
                slab_v.at[buf], gsem.at[buf])

        @pl.when(rc >= _LAST_RC)
        def _():
            pltpu.async_copy(tail_hbm, slab_v.at[buf], gsem.at[buf])

    for k in range(_NBUF - 1):
        @pl.when(k < nrun)
        def _(k=k):
            fetch(k, k)

    def run_body(r, _):
        buf = lax.rem(r, _NBUF)
        r2 = r + _NBUF - 1

        @pl.when(r2 < nrun)
        def _():
            fetch(r2, lax.rem(r2, _NBUF))

        pltpu.make_async_copy(
            tableT_hbm.at[:, pl.ds(0, _LANES)], slab_v.at[buf], gsem.at[buf]
        ).wait()

        def row_body(q, _):
            par = lax.rem(q // _BLK, 4)
            qm = lax.rem(q, _BLK)

            @pl.when(jnp.logical_and(qm == 0, q >= 4 * _BLK))
            def _():
                pltpu.make_async_copy(
                    rbuf.at[0], out_hbm.at[pl.ds(0, _BLK), :], osem
                ).wait()

            g = pl.multiple_of((q // 16) * 16, 8)
            grp_rl = s_v[pl.ds(g, 16)] & (_LANES - 1)
            rlv = jnp.take(grp_rl, jnp.full((16,), q - g, jnp.int32))
            for k in range(_DIM // 16):
                vals = plsc.load_gather(
                    slab_v.at[buf], [iota16 + (16 * k), rlv])
                rbuf[par, qm, pl.ds(16 * k, 16)] = vals

            @pl.when(qm == _BLK - 1)
            def _():
                pltpu.async_copy(
                    rbuf.at[par],
                    out_hbm.at[ord_v.at[q // _BLK]],
                    osem)

            return 0

        lax.fori_loop(_sload(runs_v, r), _sload(runs_v, r + 1), row_body, 0)
        return 0

    lax.fori_loop(0, nrun, run_body, 0)
    for _ in range(4):
        pltpu.make_async_copy(
            rbuf.at[0], out_hbm.at[pl.ds(0, _BLK), :], osem
        ).wait()


def kernel(idx, stored_mean):
    sorted_idx, ord_ = lax.sort(
        (idx, jnp.arange(_BATCH, dtype=jnp.int32)), num_keys=1)
    ord2 = ord_.reshape(_NW * _NBLK, _BLK)
    tableT = stored_mean.T
    tail = jnp.pad(
        tableT[:, _LAST_RC * _LANES:],
        ((0, 0), (0, _LAST_RC * _LANES + _LANES - _NUM_CENTERS)))
    out128 = _gather1(sorted_idx, ord2, tableT, tail)
    return out128[:, :_DIM]
